# Initial kernel scaffold; baseline (speedup 1.0000x reference)
import jax
import jax.numpy as jnp
from jax.experimental import pallas as pl


def kernel(x_seps, y_seps):
    raise NotImplementedError("write your pallas kernel here")


# speedup vs baseline: 12.3967x; 12.3967x over previous
import jax
import jax.numpy as jnp
from jax import lax
from jax.experimental import pallas as pl

LAT = 128
_B = 64


def _fill_body(xs_ref, ys_ref, out_ref):
    x = xs_ref[0, 0, :].astype(jnp.int32)
    y = ys_ref[0, 0, :].astype(jnp.int32)
    flat = y * LAT + x
    pos = lax.broadcasted_iota(jnp.int32, (_B, LAT, LAT), 1) * LAT + \
        lax.broadcasted_iota(jnp.int32, (_B, LAT, LAT), 2)
    tgt = flat[:, None, None]
    base = jnp.where(pos == 0, 1.0, 0.0)
    out = jnp.where(pos == tgt, -1.0, base)
    out_ref[...] = out.reshape(_B, 1, LAT, LAT)


def kernel(x_seps, y_seps):
    n = x_seps.shape[0]
    g = n // _B
    xs3 = x_seps.reshape(g, 1, _B)
    ys3 = y_seps.reshape(g, 1, _B)
    return pl.pallas_call(
        _fill_body,
        grid=(g,),
        in_specs=[
            pl.BlockSpec((1, 1, _B), lambda i: (i, 0, 0)),
            pl.BlockSpec((1, 1, _B), lambda i: (i, 0, 0)),
        ],
        out_specs=pl.BlockSpec((_B, 1, LAT, LAT), lambda i: (i, 0, 0, 0)),
        out_shape=jax.ShapeDtypeStruct((n, 1, LAT, LAT), jnp.float32),
    )(xs3, ys3)
